# Initial kernel scaffold; baseline (speedup 1.0000x reference)
import jax
import jax.numpy as jnp
from jax.experimental import pallas as pl


def kernel(m_prev, z_prev, x_prev, linear_w, linear_b, ln_m_w, ln_m_b, ln_z_w, ln_z_b):
    raise NotImplementedError("write your pallas kernel here")


# speedup vs baseline: 1.1622x; 1.1622x over previous
import functools

import jax
import jax.numpy as jnp
from jax.experimental import pallas as pl
from jax.experimental.pallas import tpu as pltpu

BIN_START = 3.25
BIN_END = 20.75
BIN_COUNT = 15
N_RES = 384
C_Z = 128
C_M = 256
EPS = 1e-5

BR = 8


def _m_body(m_ref, w_ref, b_ref, out_ref):
    m = m_ref[...]
    mu = jnp.mean(m, axis=-1, keepdims=True)
    var = jnp.mean((m - mu) ** 2, axis=-1, keepdims=True)
    out_ref[...] = (m - mu) * jax.lax.rsqrt(var + EPS) * w_ref[...] + b_ref[...]


def _z_body(x_smem, xv_ref, sq_ref, up_ref, w_ref, wz_ref, bz_ref, bl_ref,
            z_ref, out_ref):
    r0 = pl.program_id(0) * BR
    xj0 = xv_ref[:, 0:1]
    xj1 = xv_ref[:, 1:2]
    xj2 = xv_ref[:, 2:3]
    sq = sq_ref[...]
    up = up_ref[...]
    w = w_ref[...]
    wz = wz_ref[...]
    bz = bz_ref[...]
    bl = bl_ref[...]
    for rr in range(BR):
        i = r0 + rr
        d0 = xj0 - x_smem[i, 0]
        d1 = xj1 - x_smem[i, 1]
        d2 = xj2 - x_smem[i, 2]
        d2c = d0 * d0 + d1 * d1 + d2 * d2
        oh = ((d2c > sq) & (d2c < up)).astype(jnp.float32)
        emb = jnp.dot(oh, w, preferred_element_type=jnp.float32)
        zrow = z_ref[rr]
        mu = jnp.mean(zrow, axis=-1, keepdims=True)
        var = jnp.mean((zrow - mu) ** 2, axis=-1, keepdims=True)
        zn = (zrow - mu) * jax.lax.rsqrt(var + EPS) * wz + bz
        out_ref[rr] = zn + emb + bl


def kernel(m_prev, z_prev, x_prev, linear_w, linear_b,
           ln_m_w, ln_m_b, ln_z_w, ln_z_b):
    m_row = m_prev[0, 0]
    z = z_prev[0]
    x = x_prev[0]

    bins = jnp.linspace(BIN_START, BIN_END, BIN_COUNT, dtype=jnp.float32)
    sq = (bins ** 2).reshape(1, BIN_COUNT)
    up = jnp.concatenate(
        [sq[:, 1:], jnp.full((1, 1), jnp.inf, dtype=jnp.float32)], axis=1)

    m_out = pl.pallas_call(
        _m_body,
        out_shape=jax.ShapeDtypeStruct((N_RES, C_M), jnp.float32),
    )(m_row, ln_m_w.reshape(1, C_M), ln_m_b.reshape(1, C_M))

    grid = (N_RES // BR,)
    z_out = pl.pallas_call(
        _z_body,
        grid=grid,
        in_specs=[
            pl.BlockSpec(memory_space=pltpu.SMEM),
            pl.BlockSpec((N_RES, 3), lambda i: (0, 0)),
            pl.BlockSpec((1, BIN_COUNT), lambda i: (0, 0)),
            pl.BlockSpec((1, BIN_COUNT), lambda i: (0, 0)),
            pl.BlockSpec((BIN_COUNT, C_Z), lambda i: (0, 0)),
            pl.BlockSpec((1, C_Z), lambda i: (0, 0)),
            pl.BlockSpec((1, C_Z), lambda i: (0, 0)),
            pl.BlockSpec((1, C_Z), lambda i: (0, 0)),
            pl.BlockSpec((BR, N_RES, C_Z), lambda i: (i, 0, 0)),
        ],
        out_specs=pl.BlockSpec((BR, N_RES, C_Z), lambda i: (i, 0, 0)),
        out_shape=jax.ShapeDtypeStruct((N_RES, N_RES, C_Z), jnp.float32),
        compiler_params=pltpu.CompilerParams(
            dimension_semantics=("arbitrary",)),
    )(x, x, sq, up, linear_w,
      ln_z_w.reshape(1, C_Z), ln_z_b.reshape(1, C_Z),
      linear_b.reshape(1, C_Z), z)

    return (m_out[None], z_out[None])


# speedup vs baseline: 1.1758x; 1.0117x over previous
import jax
import jax.numpy as jnp
from jax.experimental import pallas as pl
from jax.experimental.pallas import tpu as pltpu

BIN_START = 3.25
BIN_END = 20.75
BIN_COUNT = 15
N_RES = 384
C_Z = 128
C_M = 256
EPS = 1e-5

BR = 8


def _m_body(m_ref, w_ref, b_ref, out_ref):
    m = m_ref[...]
    mu = jnp.mean(m, axis=-1, keepdims=True)
    var = jnp.mean((m - mu) ** 2, axis=-1, keepdims=True)
    out_ref[...] = (m - mu) * jax.lax.rsqrt(var + EPS) * w_ref[...] + b_ref[...]


def _idx_body(xv_ref, xt_ref, sq_ref, out_ref):
    acc = jnp.zeros((N_RES, N_RES), jnp.float32)
    d2 = jnp.zeros((N_RES, N_RES), jnp.float32)
    for c in range(3):
        diff = xv_ref[:, c:c + 1] - xt_ref[c:c + 1, :]
        d2 = d2 + diff * diff
    for i in range(BIN_COUNT):
        acc = acc + (d2 > sq_ref[0:1, i:i + 1]).astype(jnp.float32)
    out_ref[...] = acc.astype(jnp.int32)


def _z_body(idx_ref, cmat_ref, ones_ref, iota_ref, wpad_ref, wz_ref,
            z_ref, out_ref):
    zb = z_ref[...].reshape(BR * N_RES, C_Z)
    zbc = jnp.dot(zb, cmat_ref[...], preferred_element_type=jnp.float32)
    e2 = jnp.dot(zbc * zbc, ones_ref[...], preferred_element_type=jnp.float32)
    inv = jax.lax.rsqrt(e2 + EPS)
    g = (idx_ref[...] == iota_ref[...]).astype(jnp.float32)
    emb = jnp.dot(g, wpad_ref[...], preferred_element_type=jnp.float32)
    out = zbc * inv * wz_ref[...] + emb
    out_ref[...] = out.reshape(BR, N_RES, C_Z)


def kernel(m_prev, z_prev, x_prev, linear_w, linear_b,
           ln_m_w, ln_m_b, ln_z_w, ln_z_b):
    m_row = m_prev[0, 0]
    z = z_prev[0]
    x = x_prev[0]

    bins = jnp.linspace(BIN_START, BIN_END, BIN_COUNT, dtype=jnp.float32)
    sq = (bins ** 2).reshape(1, BIN_COUNT)

    m_out = pl.pallas_call(
        _m_body,
        out_shape=jax.ShapeDtypeStruct((N_RES, C_M), jnp.float32),
    )(m_row, ln_m_w.reshape(1, C_M), ln_m_b.reshape(1, C_M))

    idx = pl.pallas_call(
        _idx_body,
        out_shape=jax.ShapeDtypeStruct((N_RES, N_RES), jnp.int32),
    )(x, x.T, sq)
    idx_col = idx.reshape(N_RES * N_RES, 1)

    wpad = (jnp.concatenate([jnp.zeros((1, C_Z), jnp.float32), linear_w])
            + ln_z_b[None, :] + linear_b[None, :])
    cmat = (jnp.eye(C_Z, dtype=jnp.float32)
            - jnp.full((C_Z, C_Z), 1.0 / C_Z, jnp.float32))
    iota16 = jnp.arange(BIN_COUNT + 1, dtype=jnp.int32).reshape(1, -1)

    grid = (N_RES // BR,)
    z_out = pl.pallas_call(
        _z_body,
        grid=grid,
        in_specs=[
            pl.BlockSpec((BR * N_RES, 1), lambda i: (i, 0)),
            pl.BlockSpec((C_Z, C_Z), lambda i: (0, 0)),
            pl.BlockSpec((C_Z, 1), lambda i: (0, 0)),
            pl.BlockSpec((1, BIN_COUNT + 1), lambda i: (0, 0)),
            pl.BlockSpec((BIN_COUNT + 1, C_Z), lambda i: (0, 0)),
            pl.BlockSpec((1, C_Z), lambda i: (0, 0)),
            pl.BlockSpec((BR, N_RES, C_Z), lambda i: (i, 0, 0)),
        ],
        out_specs=pl.BlockSpec((BR, N_RES, C_Z), lambda i: (i, 0, 0)),
        out_shape=jax.ShapeDtypeStruct((N_RES, N_RES, C_Z), jnp.float32),
        compiler_params=pltpu.CompilerParams(
            dimension_semantics=("arbitrary",)),
    )(idx_col, cmat, jnp.full((C_Z, 1), 1.0 / C_Z, jnp.float32),
      iota16, wpad, ln_z_w.reshape(1, C_Z), z)

    return (m_out[None], z_out[None])


# speedup vs baseline: 1.3742x; 1.1688x over previous
import jax
import jax.numpy as jnp
from jax.experimental import pallas as pl
from jax.experimental.pallas import tpu as pltpu

BIN_START = 3.25
BIN_END = 20.75
BIN_COUNT = 15
N_RES = 384
C_Z = 128
C_M = 256
EPS = 1e-5

BR = 16
P = BR * N_RES


def _m_body(m_ref, w_ref, b_ref, out_ref):
    m = m_ref[...]
    mu = jnp.mean(m, axis=-1, keepdims=True)
    var = jnp.mean((m - mu) ** 2, axis=-1, keepdims=True)
    out_ref[...] = (m - mu) * jax.lax.rsqrt(var + EPS) * w_ref[...] + b_ref[...]


def _z_body(x_smem, xv_ref, xbt_ref, sq_ref, wd_ref, ones_ref, wz_ref,
            z_ref, out_ref):
    i = pl.program_id(0)
    x0 = xv_ref[:, 0:1]
    x1 = xv_ref[:, 1:2]
    x2 = xv_ref[:, 2:3]
    xsq = x0 * x0 + x1 * x1 + x2 * x2
    neg2dot = jnp.dot(xv_ref[...], xbt_ref[0],
                      preferred_element_type=jnp.float32)
    cols = []
    for rr in range(BR):
        r = i * BR + rr
        a0 = x_smem[r, 0]
        a1 = x_smem[r, 1]
        a2 = x_smem[r, 2]
        xsqi = a0 * a0 + a1 * a1 + a2 * a2
        cols.append(neg2dot[:, rr:rr + 1] + (xsq + xsqi))
    d2all = jnp.concatenate(cols, axis=0)
    g = (d2all > sq_ref[...]).astype(jnp.float32)
    emb = jnp.dot(g, wd_ref[...], preferred_element_type=jnp.float32)
    zb = z_ref[...].reshape(P, C_Z)
    mu = jnp.dot(zb, ones_ref[...], preferred_element_type=jnp.float32)
    e2 = jnp.dot(zb * zb, ones_ref[...], preferred_element_type=jnp.float32)
    inv = jax.lax.rsqrt(e2 - mu * mu + EPS)
    out = (zb - mu) * inv * wz_ref[...] + emb
    out_ref[...] = out.reshape(BR, N_RES, C_Z)


def kernel(m_prev, z_prev, x_prev, linear_w, linear_b,
           ln_m_w, ln_m_b, ln_z_w, ln_z_b):
    m_row = m_prev[0, 0]
    z = z_prev[0]
    x = x_prev[0]

    bins = jnp.linspace(BIN_START, BIN_END, BIN_COUNT, dtype=jnp.float32)
    sq16 = jnp.concatenate(
        [jnp.full((1,), -1e30, jnp.float32), bins ** 2]).reshape(1, 16)
    wdelta = jnp.concatenate([
        (ln_z_b + linear_b).reshape(1, C_Z),
        linear_w[0:1],
        linear_w[1:] - linear_w[:-1],
    ], axis=0)
    xbt = (-2.0 * x.T).reshape(3, N_RES // BR, BR).transpose(1, 0, 2)

    m_out = pl.pallas_call(
        _m_body,
        out_shape=jax.ShapeDtypeStruct((N_RES, C_M), jnp.float32),
    )(m_row, ln_m_w.reshape(1, C_M), ln_m_b.reshape(1, C_M))

    grid = (N_RES // BR,)
    z_out = pl.pallas_call(
        _z_body,
        grid=grid,
        in_specs=[
            pl.BlockSpec(memory_space=pltpu.SMEM),
            pl.BlockSpec((N_RES, 3), lambda i: (0, 0)),
            pl.BlockSpec((1, 3, BR), lambda i: (i, 0, 0)),
            pl.BlockSpec((1, 16), lambda i: (0, 0)),
            pl.BlockSpec((16, C_Z), lambda i: (0, 0)),
            pl.BlockSpec((C_Z, 1), lambda i: (0, 0)),
            pl.BlockSpec((1, C_Z), lambda i: (0, 0)),
            pl.BlockSpec((BR, N_RES, C_Z), lambda i: (i, 0, 0)),
        ],
        out_specs=pl.BlockSpec((BR, N_RES, C_Z), lambda i: (i, 0, 0)),
        out_shape=jax.ShapeDtypeStruct((N_RES, N_RES, C_Z), jnp.float32),
        compiler_params=pltpu.CompilerParams(
            dimension_semantics=("arbitrary",)),
    )(x, x, xbt, sq16, wdelta,
      jnp.full((C_Z, 1), 1.0 / C_Z, jnp.float32),
      ln_z_w.reshape(1, C_Z), z)

    return (m_out[None], z_out[None])


# speedup vs baseline: 1.6182x; 1.1776x over previous
import jax
import jax.numpy as jnp
from jax.experimental import pallas as pl
from jax.experimental.pallas import tpu as pltpu

BIN_START = 3.25
BIN_END = 20.75
BIN_COUNT = 15
N_RES = 384
C_Z = 128
C_M = 256
EPS = 1e-5

BR = 8


def _m_body(m_ref, w_ref, b_ref, out_ref):
    m = m_ref[...]
    mu = jnp.mean(m, axis=-1, keepdims=True)
    var = jnp.mean((m - mu) ** 2, axis=-1, keepdims=True)
    out_ref[...] = (m - mu) * jax.lax.rsqrt(var + EPS) * w_ref[...] + b_ref[...]


def _z_body(xv_ref, xbt_ref, sq_ref, wd_ref, cmat_ref, ones_ref, wz_ref,
            z_ref, out_ref):
    x0 = xv_ref[:, 0:1]
    x1 = xv_ref[:, 1:2]
    x2 = xv_ref[:, 2:3]
    xsq = x0 * x0 + x1 * x1 + x2 * x2
    xb = xbt_ref[0]
    xsqi = jnp.sum(xb * xb, axis=0, keepdims=True) * 0.25
    neg2dot = jax.lax.dot(xv_ref[...], xb,
                          precision=jax.lax.Precision.HIGHEST,
                          preferred_element_type=jnp.float32)
    d2m = neg2dot + xsq + xsqi
    zb = z_ref[...].reshape(BR * N_RES, C_Z)
    zbc = jnp.dot(zb, cmat_ref[...], preferred_element_type=jnp.float32)
    e2 = jnp.dot(zbc * zbc, ones_ref[...], preferred_element_type=jnp.float32)
    inv = jax.lax.rsqrt(e2 + EPS)
    wz = wz_ref[...]
    sq16 = sq_ref[...]
    wd = wd_ref[...]
    for rr in range(BR):
        g = (d2m[:, rr:rr + 1] > sq16).astype(jnp.float32)
        emb = jnp.dot(g, wd, preferred_element_type=jnp.float32)
        s = slice(rr * N_RES, (rr + 1) * N_RES)
        out_ref[rr] = zbc[s] * inv[s] * wz + emb


def kernel(m_prev, z_prev, x_prev, linear_w, linear_b,
           ln_m_w, ln_m_b, ln_z_w, ln_z_b):
    m_row = m_prev[0, 0]
    z = z_prev[0]
    x = x_prev[0]

    bins = jnp.linspace(BIN_START, BIN_END, BIN_COUNT, dtype=jnp.float32)
    sq16 = jnp.concatenate(
        [jnp.full((1,), -1e30, jnp.float32), bins ** 2]).reshape(1, 16)
    wdelta = jnp.concatenate([
        (ln_z_b + linear_b).reshape(1, C_Z),
        linear_w[0:1],
        linear_w[1:] - linear_w[:-1],
    ], axis=0)
    cmat = (jnp.eye(C_Z, dtype=jnp.float32)
            - jnp.full((C_Z, C_Z), 1.0 / C_Z, jnp.float32))
    xbt = (-2.0 * x.T).reshape(3, N_RES // BR, BR).transpose(1, 0, 2)

    m_out = pl.pallas_call(
        _m_body,
        out_shape=jax.ShapeDtypeStruct((N_RES, C_M), jnp.float32),
    )(m_row, ln_m_w.reshape(1, C_M), ln_m_b.reshape(1, C_M))

    grid = (N_RES // BR,)
    z_out = pl.pallas_call(
        _z_body,
        grid=grid,
        in_specs=[
            pl.BlockSpec((N_RES, 3), lambda i: (0, 0)),
            pl.BlockSpec((1, 3, BR), lambda i: (i, 0, 0)),
            pl.BlockSpec((1, 16), lambda i: (0, 0)),
            pl.BlockSpec((16, C_Z), lambda i: (0, 0)),
            pl.BlockSpec((C_Z, C_Z), lambda i: (0, 0)),
            pl.BlockSpec((C_Z, 1), lambda i: (0, 0)),
            pl.BlockSpec((1, C_Z), lambda i: (0, 0)),
            pl.BlockSpec((BR, N_RES, C_Z), lambda i: (i, 0, 0)),
        ],
        out_specs=pl.BlockSpec((BR, N_RES, C_Z), lambda i: (i, 0, 0)),
        out_shape=jax.ShapeDtypeStruct((N_RES, N_RES, C_Z), jnp.float32),
        compiler_params=pltpu.CompilerParams(
            dimension_semantics=("arbitrary",)),
    )(x, xbt, sq16, wdelta, cmat,
      jnp.full((C_Z, 1), 1.0 / C_Z, jnp.float32),
      ln_z_w.reshape(1, C_Z), z)

    return (m_out[None], z_out[None])


# speedup vs baseline: 2.0144x; 1.2449x over previous
import jax
import jax.numpy as jnp
from jax.experimental import pallas as pl
from jax.experimental.pallas import tpu as pltpu

BIN_START = 3.25
BIN_END = 20.75
BIN_COUNT = 15
N_RES = 384
C_Z = 128
C_M = 256
EPS = 1e-5

BR = 16


def _m_body(m_ref, w_ref, b_ref, out_ref):
    m = m_ref[...]
    mu = jnp.mean(m, axis=-1, keepdims=True)
    var = jnp.mean((m - mu) ** 2, axis=-1, keepdims=True)
    out_ref[...] = (m - mu) * jax.lax.rsqrt(var + EPS) * w_ref[...] + b_ref[...]


def _z_body(xv_ref, xbt_ref, sq_ref, wd_ref, cmat_ref, ones_ref, wz_ref,
            z_ref, out_ref):
    x0 = xv_ref[:, 0:1]
    x1 = xv_ref[:, 1:2]
    x2 = xv_ref[:, 2:3]
    xsq = x0 * x0 + x1 * x1 + x2 * x2
    xb = xbt_ref[0]
    xsqi = jnp.sum(xb * xb, axis=0, keepdims=True) * 0.25
    neg2dot = jax.lax.dot(xv_ref[...], xb,
                          precision=jax.lax.Precision.HIGHEST,
                          preferred_element_type=jnp.float32)
    d2m = neg2dot + xsq + xsqi
    zb = z_ref[...].reshape(BR * N_RES, C_Z)
    zbc = jnp.dot(zb, cmat_ref[...], preferred_element_type=jnp.float32)
    e2 = jnp.dot(zbc * zbc, ones_ref[...], preferred_element_type=jnp.float32)
    inv = jax.lax.rsqrt(e2 + EPS)
    wz = wz_ref[...]
    sq16 = sq_ref[...]
    wd = wd_ref[...]
    for rr in range(BR):
        g = (d2m[:, rr:rr + 1] > sq16).astype(jnp.float32)
        emb = jnp.dot(g, wd, preferred_element_type=jnp.float32)
        s = slice(rr * N_RES, (rr + 1) * N_RES)
        out_ref[rr] = zbc[s] * inv[s] * wz + emb


def kernel(m_prev, z_prev, x_prev, linear_w, linear_b,
           ln_m_w, ln_m_b, ln_z_w, ln_z_b):
    m_row = m_prev[0, 0]
    z = z_prev[0]
    x = x_prev[0]

    bins = jnp.linspace(BIN_START, BIN_END, BIN_COUNT, dtype=jnp.float32)
    sq16 = jnp.concatenate(
        [jnp.full((1,), -1e30, jnp.float32), bins ** 2]).reshape(1, 16)
    wdelta = jnp.concatenate([
        (ln_z_b + linear_b).reshape(1, C_Z),
        linear_w[0:1],
        linear_w[1:] - linear_w[:-1],
    ], axis=0)
    cmat = (jnp.eye(C_Z, dtype=jnp.float32)
            - jnp.full((C_Z, C_Z), 1.0 / C_Z, jnp.float32))
    xbt = (-2.0 * x.T).reshape(3, N_RES // BR, BR).transpose(1, 0, 2)

    m_out = pl.pallas_call(
        _m_body,
        out_shape=jax.ShapeDtypeStruct((N_RES, C_M), jnp.float32),
    )(m_row, ln_m_w.reshape(1, C_M), ln_m_b.reshape(1, C_M))

    grid = (N_RES // BR,)
    z_out = pl.pallas_call(
        _z_body,
        grid=grid,
        in_specs=[
            pl.BlockSpec((N_RES, 3), lambda i: (0, 0)),
            pl.BlockSpec((1, 3, BR), lambda i: (i, 0, 0)),
            pl.BlockSpec((1, 16), lambda i: (0, 0)),
            pl.BlockSpec((16, C_Z), lambda i: (0, 0)),
            pl.BlockSpec((C_Z, C_Z), lambda i: (0, 0)),
            pl.BlockSpec((C_Z, 1), lambda i: (0, 0)),
            pl.BlockSpec((1, C_Z), lambda i: (0, 0)),
            pl.BlockSpec((BR, N_RES, C_Z), lambda i: (i, 0, 0)),
        ],
        out_specs=pl.BlockSpec((BR, N_RES, C_Z), lambda i: (i, 0, 0)),
        out_shape=jax.ShapeDtypeStruct((N_RES, N_RES, C_Z), jnp.float32),
        compiler_params=pltpu.CompilerParams(
            dimension_semantics=("arbitrary",)),
    )(x, xbt, sq16, wdelta, cmat,
      jnp.full((C_Z, 1), 1.0 / C_Z, jnp.float32),
      ln_z_w.reshape(1, C_Z), z)

    return (m_out[None], z_out[None])


# speedup vs baseline: 2.2727x; 1.1282x over previous
import jax
import jax.numpy as jnp
from jax.experimental import pallas as pl
from jax.experimental.pallas import tpu as pltpu

BIN_START = 3.25
BIN_END = 20.75
BIN_COUNT = 15
N_RES = 384
C_Z = 128
C_M = 256
EPS = 1e-5

BR = 32


def _m_body(m_ref, w_ref, b_ref, out_ref):
    m = m_ref[...]
    mu = jnp.mean(m, axis=-1, keepdims=True)
    var = jnp.mean((m - mu) ** 2, axis=-1, keepdims=True)
    out_ref[...] = (m - mu) * jax.lax.rsqrt(var + EPS) * w_ref[...] + b_ref[...]


def _z_body(xv_ref, xbt_ref, sq_ref, wd_ref, cmat_ref, ones_ref, wz_ref,
            z_ref, out_ref):
    x0 = xv_ref[:, 0:1]
    x1 = xv_ref[:, 1:2]
    x2 = xv_ref[:, 2:3]
    xsq = x0 * x0 + x1 * x1 + x2 * x2
    xb = xbt_ref[0]
    xsqi = jnp.sum(xb * xb, axis=0, keepdims=True) * 0.25
    neg2dot = jax.lax.dot(xv_ref[...], xb,
                          precision=jax.lax.Precision.HIGHEST,
                          preferred_element_type=jnp.float32)
    d2m = neg2dot + xsq + xsqi
    zb = z_ref[...].reshape(BR * N_RES, C_Z)
    zbc = jnp.dot(zb, cmat_ref[...], preferred_element_type=jnp.float32)
    e2 = jnp.dot(zbc * zbc, ones_ref[...], preferred_element_type=jnp.float32)
    inv = jax.lax.rsqrt(e2 + EPS)
    wz = wz_ref[...]
    sq16 = sq_ref[...]
    wd = wd_ref[...]
    for rr in range(BR):
        g = (d2m[:, rr:rr + 1] > sq16).astype(jnp.float32)
        emb = jnp.dot(g, wd, preferred_element_type=jnp.float32)
        s = slice(rr * N_RES, (rr + 1) * N_RES)
        out_ref[rr] = zbc[s] * inv[s] * wz + emb


def kernel(m_prev, z_prev, x_prev, linear_w, linear_b,
           ln_m_w, ln_m_b, ln_z_w, ln_z_b):
    m_row = m_prev[0, 0]
    z = z_prev[0]
    x = x_prev[0]

    bins = jnp.linspace(BIN_START, BIN_END, BIN_COUNT, dtype=jnp.float32)
    sq16 = jnp.concatenate(
        [jnp.full((1,), -1e30, jnp.float32), bins ** 2]).reshape(1, 16)
    wdelta = jnp.concatenate([
        (ln_z_b + linear_b).reshape(1, C_Z),
        linear_w[0:1],
        linear_w[1:] - linear_w[:-1],
    ], axis=0)
    cmat = (jnp.eye(C_Z, dtype=jnp.float32)
            - jnp.full((C_Z, C_Z), 1.0 / C_Z, jnp.float32))
    xbt = (-2.0 * x.T).reshape(3, N_RES // BR, BR).transpose(1, 0, 2)

    m_out = pl.pallas_call(
        _m_body,
        out_shape=jax.ShapeDtypeStruct((N_RES, C_M), jnp.float32),
    )(m_row, ln_m_w.reshape(1, C_M), ln_m_b.reshape(1, C_M))

    grid = (N_RES // BR,)
    z_out = pl.pallas_call(
        _z_body,
        grid=grid,
        in_specs=[
            pl.BlockSpec((N_RES, 3), lambda i: (0, 0)),
            pl.BlockSpec((1, 3, BR), lambda i: (i, 0, 0)),
            pl.BlockSpec((1, 16), lambda i: (0, 0)),
            pl.BlockSpec((16, C_Z), lambda i: (0, 0)),
            pl.BlockSpec((C_Z, C_Z), lambda i: (0, 0)),
            pl.BlockSpec((C_Z, 1), lambda i: (0, 0)),
            pl.BlockSpec((1, C_Z), lambda i: (0, 0)),
            pl.BlockSpec((BR, N_RES, C_Z), lambda i: (i, 0, 0)),
        ],
        out_specs=pl.BlockSpec((BR, N_RES, C_Z), lambda i: (i, 0, 0)),
        out_shape=jax.ShapeDtypeStruct((N_RES, N_RES, C_Z), jnp.float32),
        compiler_params=pltpu.CompilerParams(
            dimension_semantics=("arbitrary",)),
    )(x, xbt, sq16, wdelta, cmat,
      jnp.full((C_Z, 1), 1.0 / C_Z, jnp.float32),
      ln_z_w.reshape(1, C_Z), z)

    return (m_out[None], z_out[None])
